# Initial kernel scaffold; baseline (speedup 1.0000x reference)
import jax
import jax.numpy as jnp
from jax.experimental import pallas as pl


def kernel(indices, values, x):
    raise NotImplementedError("write your pallas kernel here")


# speedup vs baseline: 2.1852x; 2.1852x over previous
import jax
import jax.numpy as jnp
from jax import lax
from jax.experimental import pallas as pl
from jax.experimental.pallas import tpu as pltpu
from jax.experimental.pallas import tpu_sc as plsc

N_NODES = 10000
N_EDGES = 320000
D = 128
DH = D // 2

NC = 2
NS = 16
NW = NC * NS
CHUNK = 128
NCH = 80
EPT = NCH * CHUNK
E_PAD = NW * EPT
N_PAD = 10240
ROWS_PER_TILE = N_PAD // NS
ZROWS = 128


def _sc_body(row_hbm, col_hbm, val_hbm, xa_hbm, xb_hbm, out_hbm,
             rowi_v, coli_v, vals_v, rows_v, zbuf_v, acc_sh, sem):
    c = lax.axis_index("c")
    s = lax.axis_index("s")
    tbase = (c * NS + s) * NCH

    pltpu.sync_copy(row_hbm.at[pl.ds(tbase, NCH)], rowi_v)
    pltpu.sync_copy(col_hbm.at[pl.ds(tbase, NCH)], coli_v)
    pltpu.sync_copy(val_hbm.at[pl.ds(tbase, NCH)], vals_v)

    zero = jnp.zeros((16,), jnp.float32)

    def zrow(e, _):
        for j in range(DH // 16):
            zbuf_v[e, pl.ds(j * 16, 16)] = zero
        return 0

    lax.fori_loop(0, ZROWS, zrow, 0)

    for p, x_hbm in enumerate((xa_hbm, xb_hbm)):
        for k in range(ROWS_PER_TILE // ZROWS):
            pltpu.sync_copy(zbuf_v,
                            acc_sh.at[pl.ds(s * ROWS_PER_TILE + k * ZROWS, ZROWS)])
        plsc.subcore_barrier()

        def chunk_body(ci, _):
            pltpu.async_copy(x_hbm.at[coli_v.at[ci]], rows_v, sem).wait()

            def group_body(g, _):
                v16 = vals_v[ci, pl.ds(g * 16, 16)]
                base = g * 16
                for e16 in range(16):
                    v = v16[e16]
                    for j in range(DH // 16):
                        rows_v[base + e16, pl.ds(j * 16, 16)] = (
                            rows_v[base + e16, pl.ds(j * 16, 16)] * v)
                return 0

            lax.fori_loop(0, CHUNK // 16, group_body, 0)
            pltpu.sync_copy(rows_v, acc_sh.at[rowi_v.at[ci]], add=True)
            return 0

        lax.fori_loop(0, NCH, chunk_body, 0)

        plsc.subcore_barrier()
        pltpu.sync_copy(acc_sh.at[pl.ds(s * ROWS_PER_TILE, ROWS_PER_TILE)],
                        out_hbm.at[c, p, pl.ds(s * ROWS_PER_TILE, ROWS_PER_TILE)])
        plsc.subcore_barrier()


def _combine_body(p_ref, o_ref):
    o_ref[:, :DH] = p_ref[0, 0] + p_ref[1, 0]
    o_ref[:, DH:] = p_ref[0, 1] + p_ref[1, 1]


@jax.jit
def kernel(indices, values, x):
    pad = E_PAD - N_EDGES
    row = jnp.pad(indices[0].astype(jnp.int32), (0, pad)).reshape(E_PAD // CHUNK, CHUNK)
    col = jnp.pad(indices[1].astype(jnp.int32), (0, pad)).reshape(E_PAD // CHUNK, CHUNK)
    val = jnp.pad(values, (0, pad)).reshape(E_PAD // CHUNK, CHUNK)
    xa = x[:, :DH]
    xb = x[:, DH:]

    partials = pl.kernel(
        _sc_body,
        out_type=jax.ShapeDtypeStruct((NC, 2, N_PAD, DH), jnp.float32),
        mesh=plsc.VectorSubcoreMesh(core_axis_name="c", subcore_axis_name="s",
                                    num_cores=NC, num_subcores=NS),
        scratch_types=[
            pltpu.VMEM((NCH, CHUNK), jnp.int32),
            pltpu.VMEM((NCH, CHUNK), jnp.int32),
            pltpu.VMEM((NCH, CHUNK), jnp.float32),
            pltpu.VMEM((CHUNK, DH), jnp.float32),
            pltpu.VMEM((ZROWS, DH), jnp.float32),
            pltpu.VMEM_SHARED((N_PAD, DH), jnp.float32),
            pltpu.SemaphoreType.DMA,
        ],
        compiler_params=pltpu.CompilerParams(use_tc_tiling_on_sc=False),
    )(row, col, val, xa, xb)

    blk = 1000
    return pl.pallas_call(
        _combine_body,
        grid=(N_NODES // blk,),
        in_specs=[pl.BlockSpec((NC, 2, blk, DH), lambda i: (0, 0, i, 0))],
        out_specs=pl.BlockSpec((blk, D), lambda i: (i, 0)),
        out_shape=jax.ShapeDtypeStruct((N_NODES, D), jnp.float32),
    )(partials)
